# Initial kernel scaffold; baseline (speedup 1.0000x reference)
import jax
import jax.numpy as jnp
from jax.experimental import pallas as pl


def kernel(x, edge_index, number_of_drugs, W1, b1, W2, b2, P):
    raise NotImplementedError("write your pallas kernel here")


# speedup vs baseline: 29.1302x; 29.1302x over previous
import functools

import jax
import jax.numpy as jnp
from jax import lax
from jax.experimental import pallas as pl
from jax.experimental.pallas import tpu as pltpu
from jax.experimental.pallas import tpu_sc as plsc

NC = 2
NS = 16
NW = NC * NS
C = 80
ND = 1000


def _sc_mesh():
    return plsc.VectorSubcoreMesh(
        core_axis_name="c", subcore_axis_name="s", num_cores=NC, num_subcores=NS
    )


@functools.cache
def _deg_pass(n, h, nchunk):
    rpt = n // NS

    @functools.partial(
        pl.kernel,
        out_type=jax.ShapeDtypeStruct((NC, n, h), jnp.float32),
        mesh=_sc_mesh(),
        scratch_types=[
            pltpu.VMEM((nchunk, C), jnp.int32),
            pltpu.VMEM((C, h), jnp.float32),
            pltpu.VMEM_SHARED((n, h), jnp.float32),
        ],
        compiler_params=pltpu.CompilerParams(use_tc_tiling_on_sc=False),
    )
    def deg_pass(dst_hbm, zeros_hbm, ones_hbm, out_hbm, dst_v, buf, acc):
        cid = lax.axis_index("c")
        sid = lax.axis_index("s")
        wid = sid * NC + cid
        pltpu.sync_copy(dst_hbm.at[wid], dst_v)
        pltpu.sync_copy(ones_hbm, buf)
        pltpu.sync_copy(
            zeros_hbm.at[pl.ds(sid * rpt, rpt)], acc.at[pl.ds(sid * rpt, rpt)]
        )
        plsc.subcore_barrier()

        def body(j, carry):
            pltpu.sync_copy(buf, acc.at[dst_v.at[j]], add=True)
            return carry

        lax.fori_loop(0, nchunk, body, 0)
        plsc.subcore_barrier()
        pltpu.sync_copy(
            acc.at[pl.ds(sid * rpt, rpt)], out_hbm.at[cid, pl.ds(sid * rpt, rpt)]
        )

    return deg_pass


@functools.cache
def _edge_pass(n, h, nchunk):
    rpt = n // NS

    @functools.partial(
        pl.kernel,
        out_type=jax.ShapeDtypeStruct((NC, n, h), jnp.float32),
        mesh=_sc_mesh(),
        scratch_types=[
            pltpu.VMEM((nchunk, C), jnp.int32),
            pltpu.VMEM((nchunk, C), jnp.int32),
            pltpu.VMEM((C, h), jnp.float32),
            pltpu.VMEM_SHARED((n, h), jnp.float32),
            pltpu.SemaphoreType.DMA,
        ],
        compiler_params=pltpu.CompilerParams(use_tc_tiling_on_sc=False),
    )
    def edge_pass(g_hbm, src_hbm, dst_hbm, zeros_hbm, out_hbm,
                  src_v, dst_v, buf, acc, sem):
        cid = lax.axis_index("c")
        sid = lax.axis_index("s")
        wid = sid * NC + cid
        pltpu.sync_copy(src_hbm.at[wid], src_v)
        pltpu.sync_copy(dst_hbm.at[wid], dst_v)
        pltpu.sync_copy(
            zeros_hbm.at[pl.ds(sid * rpt, rpt)], acc.at[pl.ds(sid * rpt, rpt)]
        )
        plsc.subcore_barrier()

        def body(j, carry):
            pltpu.async_copy(g_hbm.at[src_v.at[j]], buf, sem).wait()
            pltpu.sync_copy(buf, acc.at[dst_v.at[j]], add=True)
            return carry

        lax.fori_loop(0, nchunk, body, 0)
        plsc.subcore_barrier()
        pltpu.sync_copy(
            acc.at[pl.ds(sid * rpt, rpt)], out_hbm.at[cid, pl.ds(sid * rpt, rpt)]
        )

    return edge_pass


def _combine1_body(dd_ref, x_ref, w1_ref, g1_ref, dinv_ref):
    deg = dd_ref[0] + dd_ref[1] + 1.0
    dinv = lax.rsqrt(deg)
    g = jnp.dot(x_ref[...], w1_ref[...], preferred_element_type=jnp.float32)
    g1_ref[...] = g * dinv
    dinv_ref[...] = dinv


def _combine2_body(p_ref, g1_ref, dinv_ref, b1_ref, w2_ref, g2_ref):
    s = (p_ref[0] + p_ref[1] + g1_ref[...]) * dinv_ref[...]
    h1 = jnp.maximum(s + b1_ref[...], 0.0)
    g2_ref[...] = (
        jnp.dot(h1, w2_ref[...], preferred_element_type=jnp.float32)
        * dinv_ref[...]
    )


def _score_body(qa_ref, qb_ref, g2_ref, dinv_ref, b2_ref, p_ref, out_ref):
    hd = (qa_ref[...] + qb_ref[...] + g2_ref[...]) * dinv_ref[...] + b2_ref[...]
    hp = jnp.dot(hd, p_ref[...], preferred_element_type=jnp.float32)
    out_ref[...] = lax.dot_general(
        hp, hd, (((1,), (1,)), ((), ())), preferred_element_type=jnp.float32
    )


def kernel(x, edge_index, number_of_drugs, W1, b1, W2, b2, P):
    n, d = x.shape
    h = W1.shape[1]
    e = edge_index.shape[1]
    ew = e // NW
    nchunk = ew // C
    assert ew * NW == e and nchunk * C == ew and n % NS == 0

    src = edge_index[0].reshape(NW, nchunk, C)
    dst = edge_index[1].reshape(NW, nchunk, C)
    zeros = jnp.zeros((n, h), jnp.float32)
    ones_c = jnp.ones((C, h), jnp.float32)

    dd = _deg_pass(n, h, nchunk)(dst, zeros, ones_c)

    bn = 1000
    grid = n // bn
    row_spec = pl.BlockSpec((bn, h), lambda i: (i, 0))
    pair_spec = pl.BlockSpec((NC, bn, h), lambda i: (0, i, 0))
    g1, dinv16 = pl.pallas_call(
        _combine1_body,
        grid=(grid,),
        in_specs=[
            pair_spec,
            pl.BlockSpec((bn, d), lambda i: (i, 0)),
            pl.BlockSpec((d, h), lambda i: (0, 0)),
        ],
        out_specs=[row_spec, row_spec],
        out_shape=[
            jax.ShapeDtypeStruct((n, h), jnp.float32),
            jax.ShapeDtypeStruct((n, h), jnp.float32),
        ],
    )(dd, x, W1)

    p1 = _edge_pass(n, h, nchunk)(g1, src, dst, zeros)

    g2 = pl.pallas_call(
        _combine2_body,
        grid=(grid,),
        in_specs=[
            pair_spec,
            row_spec,
            row_spec,
            pl.BlockSpec((1, h), lambda i: (0, 0)),
            pl.BlockSpec((h, h), lambda i: (0, 0)),
        ],
        out_specs=row_spec,
        out_shape=jax.ShapeDtypeStruct((n, h), jnp.float32),
    )(p1, g1, dinv16, b1.reshape(1, h), W2)

    p2 = _edge_pass(n, h, nchunk)(g2, src, dst, zeros)

    start = number_of_drugs - ND
    qa = lax.dynamic_slice_in_dim(p2[0], start, ND, axis=0)
    qb = lax.dynamic_slice_in_dim(p2[1], start, ND, axis=0)
    g2d = lax.dynamic_slice_in_dim(g2, start, ND, axis=0)
    dinvd = lax.dynamic_slice_in_dim(dinv16, start, ND, axis=0)

    scores = pl.pallas_call(
        _score_body,
        out_shape=jax.ShapeDtypeStruct((ND, ND), jnp.float32),
    )(qa, qb, g2d, dinvd, b2.reshape(1, h), P)
    return scores


# speedup vs baseline: 60.0423x; 2.0612x over previous
import functools

import jax
import jax.numpy as jnp
from jax import lax
from jax.experimental import pallas as pl
from jax.experimental.pallas import tpu as pltpu
from jax.experimental.pallas import tpu_sc as plsc

NC = 2
NS = 16
NW = NC * NS
C = 125
NB = 8
ND = 1000


def _sc_mesh():
    return plsc.VectorSubcoreMesh(
        core_axis_name="c", subcore_axis_name="s", num_cores=NC, num_subcores=NS
    )


@functools.cache
def _deg_pass(n, h, nchunk):
    rpt = n // NS

    @functools.partial(
        pl.kernel,
        out_type=jax.ShapeDtypeStruct((NC, n, h), jnp.float32),
        mesh=_sc_mesh(),
        scratch_types=[
            pltpu.VMEM((nchunk, C), jnp.int32),
            pltpu.VMEM((C, h), jnp.float32),
            pltpu.VMEM_SHARED((n, h), jnp.float32),
        ]
        + [pltpu.SemaphoreType.DMA] * NB,
        compiler_params=pltpu.CompilerParams(use_tc_tiling_on_sc=False),
    )
    def deg_pass(dst_hbm, zeros_hbm, ones_hbm, out_hbm, dst_v, buf, acc, *ssem):
        cid = lax.axis_index("c")
        sid = lax.axis_index("s")
        wid = sid * NC + cid
        pltpu.sync_copy(dst_hbm.at[wid], dst_v)
        pltpu.sync_copy(ones_hbm, buf)
        pltpu.sync_copy(
            zeros_hbm.at[pl.ds(sid * rpt, rpt)], acc.at[pl.ds(sid * rpt, rpt)]
        )
        plsc.subcore_barrier()

        for b in range(NB):
            pltpu.async_copy(buf, acc.at[dst_v.at[b]], ssem[b], add=True)

        def body(i, carry):
            for b in range(NB):
                j = (i + 1) * NB + b
                pltpu.make_async_copy(ones_hbm, buf, ssem[b]).wait()
                pltpu.async_copy(buf, acc.at[dst_v.at[j]], ssem[b], add=True)
            return carry

        lax.fori_loop(0, nchunk // NB - 1, body, 0)
        for b in range(NB):
            pltpu.make_async_copy(ones_hbm, buf, ssem[b]).wait()
        plsc.subcore_barrier()
        pltpu.sync_copy(
            acc.at[pl.ds(sid * rpt, rpt)], out_hbm.at[cid, pl.ds(sid * rpt, rpt)]
        )

    return deg_pass


@functools.cache
def _edge_pass(n, h, nchunk):
    rpt = n // NS

    @functools.partial(
        pl.kernel,
        out_type=jax.ShapeDtypeStruct((NC, n, h), jnp.float32),
        mesh=_sc_mesh(),
        scratch_types=[
            pltpu.VMEM((nchunk, C), jnp.int32),
            pltpu.VMEM((nchunk, C), jnp.int32),
        ]
        + [pltpu.VMEM((C, h), jnp.float32)] * NB
        + [pltpu.VMEM_SHARED((n, h), jnp.float32)]
        + [pltpu.SemaphoreType.DMA] * NB,
        compiler_params=pltpu.CompilerParams(use_tc_tiling_on_sc=False),
    )
    def edge_pass(g_hbm, src_hbm, dst_hbm, zeros_hbm, out_hbm,
                  src_v, dst_v, *rest):
        bufs = rest[:NB]
        acc = rest[NB]
        gsem = rest[NB + 1:]
        cid = lax.axis_index("c")
        sid = lax.axis_index("s")
        wid = sid * NC + cid
        pltpu.sync_copy(src_hbm.at[wid], src_v)
        pltpu.sync_copy(dst_hbm.at[wid], dst_v)
        pltpu.sync_copy(
            zeros_hbm.at[pl.ds(sid * rpt, rpt)], acc.at[pl.ds(sid * rpt, rpt)]
        )
        plsc.subcore_barrier()

        for b in range(NB):
            pltpu.async_copy(g_hbm.at[src_v.at[b]], bufs[b], gsem[b])

        def body(i, carry):
            for b in range(NB):
                j = i * NB + b
                pltpu.make_async_copy(
                    zeros_hbm.at[pl.ds(0, C)], bufs[b], gsem[b]
                ).wait()
                pltpu.sync_copy(bufs[b], acc.at[dst_v.at[j]], add=True)
                jn = jnp.where(j + NB < nchunk, j + NB, 0)
                pltpu.async_copy(g_hbm.at[src_v.at[jn]], bufs[b], gsem[b])
            return carry

        lax.fori_loop(0, nchunk // NB, body, 0)
        for b in range(NB):
            pltpu.make_async_copy(
                zeros_hbm.at[pl.ds(0, C)], bufs[b], gsem[b]
            ).wait()
        plsc.subcore_barrier()
        pltpu.sync_copy(
            acc.at[pl.ds(sid * rpt, rpt)], out_hbm.at[cid, pl.ds(sid * rpt, rpt)]
        )

    return edge_pass


def _combine1_body(dd_ref, x_ref, w1_ref, g1_ref, dinv_ref):
    deg = dd_ref[0] + dd_ref[1] + 1.0
    dinv = lax.rsqrt(deg)
    g = jnp.dot(x_ref[...], w1_ref[...], preferred_element_type=jnp.float32)
    g1_ref[...] = g * dinv
    dinv_ref[...] = dinv


def _combine2_body(p_ref, g1_ref, dinv_ref, b1_ref, w2_ref, g2_ref):
    s = (p_ref[0] + p_ref[1] + g1_ref[...]) * dinv_ref[...]
    h1 = jnp.maximum(s + b1_ref[...], 0.0)
    g2_ref[...] = (
        jnp.dot(h1, w2_ref[...], preferred_element_type=jnp.float32)
        * dinv_ref[...]
    )


def _score_body(qa_ref, qb_ref, g2_ref, dinv_ref, b2_ref, p_ref, out_ref):
    hd = (qa_ref[...] + qb_ref[...] + g2_ref[...]) * dinv_ref[...] + b2_ref[...]
    hp = jnp.dot(hd, p_ref[...], preferred_element_type=jnp.float32)
    out_ref[...] = lax.dot_general(
        hp, hd, (((1,), (1,)), ((), ())), preferred_element_type=jnp.float32
    )


def kernel(x, edge_index, number_of_drugs, W1, b1, W2, b2, P):
    n, d = x.shape
    h = W1.shape[1]
    e = edge_index.shape[1]
    ew = e // NW
    nchunk = ew // C
    assert ew * NW == e and nchunk * C == ew and n % NS == 0
    assert nchunk % NB == 0

    src = edge_index[0].reshape(NW, nchunk, C)
    dst = edge_index[1].reshape(NW, nchunk, C)
    zeros = jnp.zeros((n, h), jnp.float32)
    ones_c = jnp.ones((C, h), jnp.float32)

    dd = _deg_pass(n, h, nchunk)(dst, zeros, ones_c)

    bn = 1000
    grid = n // bn
    row_spec = pl.BlockSpec((bn, h), lambda i: (i, 0))
    pair_spec = pl.BlockSpec((NC, bn, h), lambda i: (0, i, 0))
    g1, dinv16 = pl.pallas_call(
        _combine1_body,
        grid=(grid,),
        in_specs=[
            pair_spec,
            pl.BlockSpec((bn, d), lambda i: (i, 0)),
            pl.BlockSpec((d, h), lambda i: (0, 0)),
        ],
        out_specs=[row_spec, row_spec],
        out_shape=[
            jax.ShapeDtypeStruct((n, h), jnp.float32),
            jax.ShapeDtypeStruct((n, h), jnp.float32),
        ],
    )(dd, x, W1)

    p1 = _edge_pass(n, h, nchunk)(g1, src, dst, zeros)

    g2 = pl.pallas_call(
        _combine2_body,
        grid=(grid,),
        in_specs=[
            pair_spec,
            row_spec,
            row_spec,
            pl.BlockSpec((1, h), lambda i: (0, 0)),
            pl.BlockSpec((h, h), lambda i: (0, 0)),
        ],
        out_specs=row_spec,
        out_shape=jax.ShapeDtypeStruct((n, h), jnp.float32),
    )(p1, g1, dinv16, b1.reshape(1, h), W2)

    p2 = _edge_pass(n, h, nchunk)(g2, src, dst, zeros)

    start = number_of_drugs - ND
    qa = lax.dynamic_slice_in_dim(p2[0], start, ND, axis=0)
    qb = lax.dynamic_slice_in_dim(p2[1], start, ND, axis=0)
    g2d = lax.dynamic_slice_in_dim(g2, start, ND, axis=0)
    dinvd = lax.dynamic_slice_in_dim(dinv16, start, ND, axis=0)

    scores = pl.pallas_call(
        _score_body,
        out_shape=jax.ShapeDtypeStruct((ND, ND), jnp.float32),
    )(qa, qb, g2d, dinvd, b2.reshape(1, h), P)
    return scores


# speedup vs baseline: 96.2689x; 1.6034x over previous
import functools

import jax
import jax.numpy as jnp
from jax import lax
from jax.experimental import pallas as pl
from jax.experimental.pallas import tpu as pltpu
from jax.experimental.pallas import tpu_sc as plsc

NC = 2
NS = 16
NW = NC * NS
C = 128
NB = 6
NCH = 78
XTRA = 4
ND = 1000


def _sc_mesh():
    return plsc.VectorSubcoreMesh(
        core_axis_name="c", subcore_axis_name="s", num_cores=NC, num_subcores=NS
    )


def _chunk_range(wid):
    return NCH * wid + jnp.minimum(wid, XTRA)


@functools.cache
def _deg_pass(n, h):
    rpt = n // NS

    @functools.partial(
        pl.kernel,
        out_type=jax.ShapeDtypeStruct((NC, n, h), jnp.float32),
        mesh=_sc_mesh(),
        scratch_types=[
            pltpu.VMEM((NCH + 1, 2, C), jnp.int32),
            pltpu.VMEM((C, h), jnp.float32),
            pltpu.VMEM_SHARED((n, h), jnp.float32),
        ]
        + [pltpu.SemaphoreType.DMA] * NB,
        compiler_params=pltpu.CompilerParams(use_tc_tiling_on_sc=False),
    )
    def deg_pass(er_hbm, zeros_hbm, ones_hbm, out_hbm, idx_v, buf, acc, *ssem):
        cid = lax.axis_index("c")
        sid = lax.axis_index("s")
        wid = sid * NC + cid
        start = _chunk_range(wid)
        pltpu.sync_copy(er_hbm.at[pl.ds(start, NCH)], idx_v.at[pl.ds(0, NCH)])

        @pl.when(wid < XTRA)
        def _():
            pltpu.sync_copy(
                er_hbm.at[pl.ds(start + NCH, 1)], idx_v.at[pl.ds(NCH, 1)]
            )

        pltpu.sync_copy(ones_hbm, buf)
        pltpu.sync_copy(
            zeros_hbm.at[pl.ds(sid * rpt, rpt)], acc.at[pl.ds(sid * rpt, rpt)]
        )
        plsc.subcore_barrier()

        for b in range(NB):
            pltpu.async_copy(buf, acc.at[idx_v.at[b, 1]], ssem[b], add=True)

        def body(i, carry):
            for b in range(NB):
                j = (i + 1) * NB + b
                pltpu.make_async_copy(ones_hbm, buf, ssem[b]).wait()
                pltpu.async_copy(buf, acc.at[idx_v.at[j, 1]], ssem[b], add=True)
            return carry

        lax.fori_loop(0, NCH // NB - 1, body, 0)
        for b in range(NB):
            pltpu.make_async_copy(ones_hbm, buf, ssem[b]).wait()

        @pl.when(wid < XTRA)
        def _():
            pltpu.sync_copy(buf, acc.at[idx_v.at[NCH, 1]], add=True)

        plsc.subcore_barrier()
        pltpu.sync_copy(
            acc.at[pl.ds(sid * rpt, rpt)], out_hbm.at[cid, pl.ds(sid * rpt, rpt)]
        )

    return deg_pass


@functools.cache
def _edge_pass(n, h):
    rpt = n // NS

    @functools.partial(
        pl.kernel,
        out_type=jax.ShapeDtypeStruct((NC, n, h), jnp.float32),
        mesh=_sc_mesh(),
        scratch_types=[
            pltpu.VMEM((NCH + 1, 2, C), jnp.int32),
        ]
        + [pltpu.VMEM((C, h), jnp.float32)] * NB
        + [pltpu.VMEM_SHARED((n, h), jnp.float32)]
        + [pltpu.SemaphoreType.DMA] * NB,
        compiler_params=pltpu.CompilerParams(use_tc_tiling_on_sc=False),
    )
    def edge_pass(g_hbm, er_hbm, zeros_hbm, out_hbm, idx_v, *rest):
        bufs = rest[:NB]
        acc = rest[NB]
        gsem = rest[NB + 1:]
        cid = lax.axis_index("c")
        sid = lax.axis_index("s")
        wid = sid * NC + cid
        start = _chunk_range(wid)
        pltpu.sync_copy(er_hbm.at[pl.ds(start, NCH)], idx_v.at[pl.ds(0, NCH)])

        @pl.when(wid < XTRA)
        def _():
            pltpu.sync_copy(
                er_hbm.at[pl.ds(start + NCH, 1)], idx_v.at[pl.ds(NCH, 1)]
            )

        pltpu.sync_copy(
            zeros_hbm.at[pl.ds(sid * rpt, rpt)], acc.at[pl.ds(sid * rpt, rpt)]
        )
        plsc.subcore_barrier()

        for b in range(NB):
            pltpu.async_copy(g_hbm.at[idx_v.at[b, 0]], bufs[b], gsem[b])

        def body(i, carry):
            for b in range(NB):
                j = i * NB + b
                pltpu.make_async_copy(
                    zeros_hbm.at[pl.ds(0, C)], bufs[b], gsem[b]
                ).wait()
                pltpu.sync_copy(bufs[b], acc.at[idx_v.at[j, 1]], add=True)
                jn = jnp.where(j + NB < NCH, j + NB, 0)
                pltpu.async_copy(g_hbm.at[idx_v.at[jn, 0]], bufs[b], gsem[b])
            return carry

        lax.fori_loop(0, NCH // NB, body, 0)
        for b in range(NB):
            pltpu.make_async_copy(
                zeros_hbm.at[pl.ds(0, C)], bufs[b], gsem[b]
            ).wait()

        @pl.when(wid < XTRA)
        def _():
            pltpu.async_copy(g_hbm.at[idx_v.at[NCH, 0]], bufs[0], gsem[0]).wait()
            pltpu.sync_copy(bufs[0], acc.at[idx_v.at[NCH, 1]], add=True)

        plsc.subcore_barrier()
        pltpu.sync_copy(
            acc.at[pl.ds(sid * rpt, rpt)], out_hbm.at[cid, pl.ds(sid * rpt, rpt)]
        )

    return edge_pass


def _mm_body(x_ref, w_ref, y_ref):
    y_ref[...] = jnp.dot(
        x_ref[...], w_ref[...], preferred_element_type=jnp.float32
    )


def _combine1_body(dd_ref, y1_ref, g1_ref, dinv_ref):
    deg = dd_ref[0] + dd_ref[1] + 1.0
    dinv = lax.rsqrt(deg)
    g1_ref[...] = y1_ref[...] * dinv
    dinv_ref[...] = dinv


def _combine2_body(p_ref, g1_ref, dinv_ref, b1_ref, w2_ref, g2_ref):
    s = (p_ref[0] + p_ref[1] + g1_ref[...]) * dinv_ref[...]
    h1 = jnp.maximum(s + b1_ref[...], 0.0)
    g2_ref[...] = (
        jnp.dot(h1, w2_ref[...], preferred_element_type=jnp.float32)
        * dinv_ref[...]
    )


def _hd_body(qa_ref, qb_ref, g2_ref, dinv_ref, b2_ref, hd_ref):
    hd_ref[...] = (
        (qa_ref[...] + qb_ref[...] + g2_ref[...]) * dinv_ref[...] + b2_ref[...]
    )


def _score_body(hd_ref, p_ref, out_ref):
    hd = hd_ref[...]
    hp = jnp.dot(hd, p_ref[...], preferred_element_type=jnp.float32)
    out_ref[...] = lax.dot_general(
        hp, hd, (((1,), (1,)), ((), ())), preferred_element_type=jnp.float32
    )


def kernel(x, edge_index, number_of_drugs, W1, b1, W2, b2, P):
    n, d = x.shape
    h = W1.shape[1]
    e = edge_index.shape[1]
    assert e == NW * NCH * C + XTRA * C and h == 16
    assert n % NS == 0 and n % 8 == 0
    nr = n * h // 128
    f32 = jnp.float32

    er = edge_index.reshape(2, e // C, C).transpose(1, 0, 2)
    zeros = jnp.zeros((n, h), f32)
    ones_c = jnp.ones((C, h), f32)
    w2_bd = jnp.kron(jnp.eye(8, dtype=f32), W2)
    b1t = jnp.tile(b1, 8).reshape(1, 128)
    b2t = jnp.tile(b2, 8).reshape(1, 128)

    dd = _deg_pass(n, h)(er, zeros, ones_c)
    dd_r = dd.reshape(NC, nr, 128)

    y1 = pl.pallas_call(
        _mm_body,
        out_shape=jax.ShapeDtypeStruct((n, h), f32),
    )(x, W1)
    y1_r = y1.reshape(nr, 128)

    g1_r, dinv_r = pl.pallas_call(
        _combine1_body,
        out_shape=[
            jax.ShapeDtypeStruct((nr, 128), f32),
            jax.ShapeDtypeStruct((nr, 128), f32),
        ],
    )(dd_r, y1_r)

    p1 = _edge_pass(n, h)(g1_r.reshape(n, h), er, zeros)

    g2_r = pl.pallas_call(
        _combine2_body,
        out_shape=jax.ShapeDtypeStruct((nr, 128), f32),
    )(p1.reshape(NC, nr, 128), g1_r, dinv_r, b1t, w2_bd)

    p2 = _edge_pass(n, h)(g2_r.reshape(n, h), er, zeros)
    p2_r = p2.reshape(NC, nr, 128)

    ndr = ND * h // 128
    roff = (number_of_drugs - ND) // 8
    qa_r = lax.dynamic_slice(p2_r, (0, roff, 0), (1, ndr, 128)).reshape(ndr, 128)
    qb_r = lax.dynamic_slice(p2_r, (1, roff, 0), (1, ndr, 128)).reshape(ndr, 128)
    g2d_r = lax.dynamic_slice(g2_r, (roff, 0), (ndr, 128))
    dinvd_r = lax.dynamic_slice(dinv_r, (roff, 0), (ndr, 128))

    hd_r = pl.pallas_call(
        _hd_body,
        out_shape=jax.ShapeDtypeStruct((ndr, 128), f32),
    )(qa_r, qb_r, g2d_r, dinvd_r, b2t)

    scores = pl.pallas_call(
        _score_body,
        out_shape=jax.ShapeDtypeStruct((ND, ND), f32),
    )(hd_r.reshape(ND, h), P)
    return scores


# speedup vs baseline: 101.5616x; 1.0550x over previous
import functools

import jax
import jax.numpy as jnp
from jax import lax
from jax.experimental import pallas as pl
from jax.experimental.pallas import tpu as pltpu
from jax.experimental.pallas import tpu_sc as plsc

NC = 2
NS = 16
NW = NC * NS
C = 128
NB = 6
NCH = 78
XTRA = 4
NP = 10240
ND = 1000


def _sc_mesh():
    return plsc.VectorSubcoreMesh(
        core_axis_name="c", subcore_axis_name="s", num_cores=NC, num_subcores=NS
    )


def _chunk_range(wid):
    return NCH * wid + jnp.minimum(wid, XTRA)


@functools.cache
def _deg_pass(n, h):
    rpt = n // NS

    @functools.partial(
        pl.kernel,
        out_type=jax.ShapeDtypeStruct((NC, n, h), jnp.float32),
        mesh=_sc_mesh(),
        scratch_types=[
            pltpu.VMEM((NCH + 1, 2, C), jnp.int32),
            pltpu.VMEM((C, h), jnp.float32),
            pltpu.VMEM_SHARED((n, h), jnp.float32),
        ]
        + [pltpu.SemaphoreType.DMA] * NB,
        compiler_params=pltpu.CompilerParams(use_tc_tiling_on_sc=False),
    )
    def deg_pass(er_hbm, zeros_hbm, ones_hbm, out_hbm, idx_v, buf, acc, *ssem):
        cid = lax.axis_index("c")
        sid = lax.axis_index("s")
        wid = sid * NC + cid
        start = _chunk_range(wid)
        pltpu.sync_copy(er_hbm.at[pl.ds(start, NCH)], idx_v.at[pl.ds(0, NCH)])

        @pl.when(wid < XTRA)
        def _():
            pltpu.sync_copy(
                er_hbm.at[pl.ds(start + NCH, 1)], idx_v.at[pl.ds(NCH, 1)]
            )

        pltpu.sync_copy(ones_hbm, buf)
        pltpu.sync_copy(
            zeros_hbm.at[pl.ds(sid * rpt, rpt)], acc.at[pl.ds(sid * rpt, rpt)]
        )
        plsc.subcore_barrier()

        for b in range(NB):
            pltpu.async_copy(buf, acc.at[idx_v.at[b, 1]], ssem[b], add=True)

        def body(i, carry):
            for b in range(NB):
                j = (i + 1) * NB + b
                pltpu.make_async_copy(ones_hbm, buf, ssem[b]).wait()
                pltpu.async_copy(buf, acc.at[idx_v.at[j, 1]], ssem[b], add=True)
            return carry

        lax.fori_loop(0, NCH // NB - 1, body, 0)
        for b in range(NB):
            pltpu.make_async_copy(ones_hbm, buf, ssem[b]).wait()

        @pl.when(wid < XTRA)
        def _():
            pltpu.sync_copy(buf, acc.at[idx_v.at[NCH, 1]], add=True)

        plsc.subcore_barrier()
        pltpu.sync_copy(
            acc.at[pl.ds(sid * rpt, rpt)], out_hbm.at[cid, pl.ds(sid * rpt, rpt)]
        )

    return deg_pass


@functools.cache
def _edge_pass(n, h):
    rpt = n // NS

    @functools.partial(
        pl.kernel,
        out_type=jax.ShapeDtypeStruct((NC, n, h), jnp.float32),
        mesh=_sc_mesh(),
        scratch_types=[
            pltpu.VMEM((NCH + 1, 2, C), jnp.int32),
        ]
        + [pltpu.VMEM((C, h), jnp.float32)] * NB
        + [pltpu.VMEM_SHARED((n, h), jnp.float32)]
        + [pltpu.SemaphoreType.DMA] * NB,
        compiler_params=pltpu.CompilerParams(use_tc_tiling_on_sc=False),
    )
    def edge_pass(g_hbm, er_hbm, zeros_hbm, out_hbm, idx_v, *rest):
        bufs = rest[:NB]
        acc = rest[NB]
        gsem = rest[NB + 1:]
        cid = lax.axis_index("c")
        sid = lax.axis_index("s")
        wid = sid * NC + cid
        start = _chunk_range(wid)
        pltpu.sync_copy(er_hbm.at[pl.ds(start, NCH)], idx_v.at[pl.ds(0, NCH)])

        @pl.when(wid < XTRA)
        def _():
            pltpu.sync_copy(
                er_hbm.at[pl.ds(start + NCH, 1)], idx_v.at[pl.ds(NCH, 1)]
            )

        pltpu.sync_copy(
            zeros_hbm.at[pl.ds(sid * rpt, rpt)], acc.at[pl.ds(sid * rpt, rpt)]
        )
        plsc.subcore_barrier()

        for b in range(NB):
            pltpu.async_copy(g_hbm.at[idx_v.at[b, 0]], bufs[b], gsem[b])

        def body(i, carry):
            for b in range(NB):
                j = i * NB + b
                pltpu.make_async_copy(
                    zeros_hbm.at[pl.ds(0, C)], bufs[b], gsem[b]
                ).wait()
                pltpu.sync_copy(bufs[b], acc.at[idx_v.at[j, 1]], add=True)
                jn = jnp.where(j + NB < NCH, j + NB, 0)
                pltpu.async_copy(g_hbm.at[idx_v.at[jn, 0]], bufs[b], gsem[b])
            return carry

        lax.fori_loop(0, NCH // NB, body, 0)
        for b in range(NB):
            pltpu.make_async_copy(
                zeros_hbm.at[pl.ds(0, C)], bufs[b], gsem[b]
            ).wait()

        @pl.when(wid < XTRA)
        def _():
            pltpu.async_copy(g_hbm.at[idx_v.at[NCH, 0]], bufs[0], gsem[0]).wait()
            pltpu.sync_copy(bufs[0], acc.at[idx_v.at[NCH, 1]], add=True)

        plsc.subcore_barrier()
        pltpu.sync_copy(
            acc.at[pl.ds(sid * rpt, rpt)], out_hbm.at[cid, pl.ds(sid * rpt, rpt)]
        )

    return edge_pass


def _mm_body(x_ref, w_ref, y_ref):
    n = x_ref.shape[0]
    y_ref[pl.ds(0, n), :] = jnp.dot(
        x_ref[...], w_ref[...], preferred_element_type=jnp.float32
    )


def _combine1_body(dd_ref, y1_ref, g1_ref, dinv_ref):
    deg = dd_ref[0] + dd_ref[1] + 1.0
    dinv = lax.rsqrt(deg)
    g1_ref[...] = y1_ref[...] * dinv
    dinv_ref[...] = dinv


def _combine2_body(p_ref, g1_ref, dinv_ref, b1_ref, w2_ref, g2_ref):
    s = (p_ref[0] + p_ref[1] + g1_ref[...]) * dinv_ref[...]
    h1 = jnp.maximum(s + b1_ref[...], 0.0)
    g2_ref[...] = (
        jnp.dot(h1, w2_ref[...], preferred_element_type=jnp.float32)
        * dinv_ref[...]
    )


def _hd_body(qa_ref, qb_ref, g2_ref, dinv_ref, b2_ref, hd_ref):
    hd_ref[...] = (
        (qa_ref[...] + qb_ref[...] + g2_ref[...]) * dinv_ref[...] + b2_ref[...]
    )


def _score_body(hd_ref, p_ref, out_ref):
    hd = hd_ref[pl.ds(0, out_ref.shape[0]), :]
    hp = jnp.dot(hd, p_ref[...], preferred_element_type=jnp.float32)
    out_ref[...] = lax.dot_general(
        hp, hd, (((1,), (1,)), ((), ())), preferred_element_type=jnp.float32
    )


def kernel(x, edge_index, number_of_drugs, W1, b1, W2, b2, P):
    n, d = x.shape
    h = W1.shape[1]
    e = edge_index.shape[1]
    assert e == NW * NCH * C + XTRA * C and h == 16
    assert n <= NP and NP % NS == 0 and (NP * h // 128) % 8 == 0
    nr = NP * h // 128
    f32 = jnp.float32

    er = edge_index.reshape(2, e // C, C).transpose(1, 0, 2)
    zeros = jnp.zeros((NP, h), f32)
    ones_c = jnp.ones((C, h), f32)
    w2_bd = jnp.kron(jnp.eye(8, dtype=f32), W2)
    b1t = jnp.tile(b1, 8).reshape(1, 128)
    b2t = jnp.tile(b2, 8).reshape(1, 128)

    dd = _deg_pass(NP, h)(er, zeros, ones_c)
    dd_r = dd.reshape(NC, nr, 128)

    y1 = pl.pallas_call(
        _mm_body,
        out_shape=jax.ShapeDtypeStruct((NP, h), f32),
    )(x, W1)
    y1_r = y1.reshape(nr, 128)

    g1_r, dinv_r = pl.pallas_call(
        _combine1_body,
        out_shape=[
            jax.ShapeDtypeStruct((nr, 128), f32),
            jax.ShapeDtypeStruct((nr, 128), f32),
        ],
    )(dd_r, y1_r)

    p1 = _edge_pass(NP, h)(g1_r.reshape(NP, h), er, zeros)

    g2_r = pl.pallas_call(
        _combine2_body,
        out_shape=jax.ShapeDtypeStruct((nr, 128), f32),
    )(p1.reshape(NC, nr, 128), g1_r, dinv_r, b1t, w2_bd)

    p2 = _edge_pass(NP, h)(g2_r.reshape(NP, h), er, zeros)
    p2_r = p2.reshape(NC, nr, 128)

    ndr = 128
    roff = (number_of_drugs - ND) // 8
    qa_r = lax.dynamic_slice(p2_r, (0, roff, 0), (1, ndr, 128)).reshape(ndr, 128)
    qb_r = lax.dynamic_slice(p2_r, (1, roff, 0), (1, ndr, 128)).reshape(ndr, 128)
    g2d_r = lax.dynamic_slice(g2_r, (roff, 0), (ndr, 128))
    dinvd_r = lax.dynamic_slice(dinv_r, (roff, 0), (ndr, 128))

    hd_r = pl.pallas_call(
        _hd_body,
        out_shape=jax.ShapeDtypeStruct((ndr, 128), f32),
    )(qa_r, qb_r, g2d_r, dinvd_r, b2t)

    scores = pl.pallas_call(
        _score_body,
        out_shape=jax.ShapeDtypeStruct((ND, ND), f32),
    )(hd_r.reshape(ndr * 8, h), P)
    return scores
